# baseline (speedup 1.0000x reference)
import functools

import numpy as np
import jax
import jax.numpy as jnp
from jax import lax
from jax.experimental import pallas as pl
from jax.experimental.pallas import tpu as pltpu
from jax.experimental.pallas import tpu_sc as plsc

_B, _T, _D = 1, 4096, 600
_H, _K = 64, 1024
_MASK_PROB = 0.1
_MASK_TIME = 400
_NUM_MASKS = 5
_SEED = 42
_N = 410
_NC, _NS = 2, 16
_NW = _NC * _NS
_RPW = 16
_NPAD = _NW * _RPW


@functools.lru_cache(maxsize=1)
def _consts():
    with jax.default_device(jax.devices("cpu")[0]):
        return _consts_impl()


def _consts_impl():
    k_mask = jax.random.fold_in(jax.random.key(_SEED), 0)
    logits = jax.random.uniform(k_mask, (_B, _T))
    randperm = jnp.argsort(logits, axis=-1).astype(jnp.float32)
    mask = randperm < (_T * _MASK_PROB)
    _rows, cols = jnp.nonzero(mask, size=_B * _N)
    cols = np.asarray(cols).astype(np.int32)

    rng = np.random.RandomState(_SEED)
    k_noise = jax.random.fold_in(jax.random.key(_SEED), 1)
    overlay_vals = np.zeros((_NPAD, _D), np.float32)
    overlay_mask = np.zeros((1, _D), np.float32)
    for i in range(_NUM_MASKS):
        idx = int(rng.randint(0, _D + 1))
        if idx + _MASK_TIME <= _D:
            noise = 0.1 * jax.random.normal(
                jax.random.fold_in(k_noise, i), (_N, _MASK_TIME), dtype=jnp.float32)
            overlay_vals[:_N, idx:idx + _MASK_TIME] = np.asarray(noise)
            overlay_mask[0, idx:idx + _MASK_TIME] = 1.0

    cols_pad = np.concatenate([cols, np.full((_NPAD - _N,), cols[-1], np.int32)])
    return (jnp.asarray(cols_pad), jnp.asarray(overlay_vals),
            jnp.asarray(overlay_mask))


_CONSTS = _consts()


def _sc_gather_body(x_hbm, out_hbm, sem):
    wid = lax.axis_index("s") * _NC + lax.axis_index("c")
    cols = np.asarray(_CONSTS[0])

    def _branch(w):
        def br():
            descs = []
            for j in range(_RPW):
                col = int(cols[w * _RPW + j])
                descs.append(pltpu.async_copy(
                    x_hbm.at[0, pl.ds(col, 1), :],
                    out_hbm.at[pl.ds(w * _RPW + j, 1), :], sem))
            for d in descs:
                d.wait()
        return br

    lax.switch(wid, [_branch(w) for w in range(_NW)])


def _sc_gather(x):
    return pl.kernel(
        _sc_gather_body,
        out_type=jax.ShapeDtypeStruct((_NPAD, _D), jnp.float32),
        mesh=plsc.VectorSubcoreMesh(core_axis_name="c", subcore_axis_name="s"),
        scratch_types=[
            pltpu.SemaphoreType.DMA,
        ],
    )(x)


def _tc_body(xg_ref, g_ref, b_ref, w_ref, cb_ref, ow_ref, ob_ref, ov_ref,
             om_ref, tout_ref, lab_ref):
    x = xg_ref[:]
    mu = jnp.mean(x, axis=1, keepdims=True)
    var = jnp.mean((x - mu) ** 2, axis=1, keepdims=True)
    y = (x - mu) / jnp.sqrt(var + 1e-5) * g_ref[:] + b_ref[:]

    ym = jnp.where(om_ref[:] > 0.5, ov_ref[:], y)
    tout_ref[:] = (jnp.sum(ym * ow_ref[:], axis=1, keepdims=True) + ob_ref[0, 0])

    hi = jax.lax.Precision.HIGHEST
    t = jax.lax.dot_general(y.astype(jnp.bfloat16),
                            w_ref[:].astype(jnp.bfloat16),
                            (((1,), (1,)), ((), ())),
                            preferred_element_type=jnp.float32)
    cb = cb_ref[:]
    tc = jax.lax.dot_general(t, cb, (((1,), (0,)), ((), ())),
                             precision=hi, preferred_element_type=jnp.float32)
    d2 = (jnp.sum(t * t, axis=1, keepdims=True) - 2.0 * tc
          + jnp.sum(cb * cb, axis=0, keepdims=True))
    dmin = jnp.min(d2, axis=1, keepdims=True)
    kidx = jax.lax.broadcasted_iota(jnp.int32, d2.shape, 1)
    lab_ref[:] = jnp.min(jnp.where(d2 <= dmin, kidx, _K), axis=1, keepdims=True)


def kernel(input_values, ln_gamma, ln_beta, proj_W, code_book, out_W, out_b):
    _cols_pad, overlay_vals, overlay_mask = _CONSTS
    xg = _sc_gather(input_values)

    tout, lab = pl.pallas_call(
        _tc_body,
        out_shape=(
            jax.ShapeDtypeStruct((_NPAD, 1), jnp.float32),
            jax.ShapeDtypeStruct((_NPAD, 1), jnp.int32),
        ),
    )(xg, ln_gamma.reshape(1, _D), ln_beta.reshape(1, _D), proj_W, code_book,
      out_W.reshape(1, _D), out_b.reshape(1, 1), overlay_vals, overlay_mask)

    return (tout[:_N], lab[:_N, 0])


# speedup vs baseline: 3.0686x; 3.0686x over previous
import functools

import numpy as np
import jax
import jax.numpy as jnp
from jax import lax
from jax.experimental import pallas as pl
from jax.experimental.pallas import tpu as pltpu

_B, _T, _D = 1, 4096, 600
_H, _K = 64, 1024
_MASK_PROB = 0.1
_MASK_TIME = 400
_NUM_MASKS = 5
_SEED = 42
_N = 410
_NPAD = 416


@functools.lru_cache(maxsize=1)
def _consts():
    with jax.default_device(jax.devices("cpu")[0]):
        return _consts_impl()


def _consts_impl():
    k_mask = jax.random.fold_in(jax.random.key(_SEED), 0)
    logits = jax.random.uniform(k_mask, (_B, _T))
    randperm = jnp.argsort(logits, axis=-1).astype(jnp.float32)
    mask = randperm < (_T * _MASK_PROB)
    _rows, cols = jnp.nonzero(mask, size=_B * _N)
    cols = np.asarray(cols).astype(np.int32)

    rng = np.random.RandomState(_SEED)
    k_noise = jax.random.fold_in(jax.random.key(_SEED), 1)
    overlay_vals = np.zeros((_NPAD, _D), np.float32)
    overlay_mask = np.zeros((1, _D), np.float32)
    for i in range(_NUM_MASKS):
        idx = int(rng.randint(0, _D + 1))
        if idx + _MASK_TIME <= _D:
            noise = 0.1 * jax.random.normal(
                jax.random.fold_in(k_noise, i), (_N, _MASK_TIME), dtype=jnp.float32)
            overlay_vals[:_N, idx:idx + _MASK_TIME] = np.asarray(noise)
            overlay_mask[0, idx:idx + _MASK_TIME] = 1.0

    cols_pad = np.concatenate([cols, np.full((_NPAD - _N,), cols[-1], np.int32)])
    return (jnp.asarray(cols_pad), jnp.asarray(overlay_vals),
            jnp.asarray(overlay_mask))


_CONSTS = _consts()


def _tc_body(x_ref, g_ref, b_ref, w_ref, cb_ref, ow_ref, ob_ref, ov_ref,
             om_ref, tout_ref, lab_ref, xg_ref):
    cols = np.asarray(_CONSTS[0])
    for i in range(_NPAD):
        col = int(cols[i])
        xg_ref[pl.ds(i, 1), :] = x_ref[0, pl.ds(col, 1), :]

    x = xg_ref[:]
    mu = jnp.mean(x, axis=1, keepdims=True)
    var = jnp.mean((x - mu) ** 2, axis=1, keepdims=True)
    y = (x - mu) / jnp.sqrt(var + 1e-5) * g_ref[:] + b_ref[:]

    ym = jnp.where(om_ref[:] > 0.5, ov_ref[:], y)
    tout_ref[:] = (jnp.sum(ym * ow_ref[:], axis=1, keepdims=True) + ob_ref[0, 0])

    t = jax.lax.dot_general(y.astype(jnp.bfloat16),
                            w_ref[:].astype(jnp.bfloat16),
                            (((1,), (1,)), ((), ())),
                            preferred_element_type=jnp.float32)
    hi = jax.lax.Precision.HIGHEST
    cb = cb_ref[:]
    tc = jax.lax.dot_general(t, cb, (((1,), (0,)), ((), ())),
                             precision=hi, preferred_element_type=jnp.float32)
    d2 = (jnp.sum(t * t, axis=1, keepdims=True) - 2.0 * tc
          + jnp.sum(cb * cb, axis=0, keepdims=True))
    dmin = jnp.min(d2, axis=1, keepdims=True)
    kidx = jax.lax.broadcasted_iota(jnp.int32, d2.shape, 1)
    lab_ref[:] = jnp.min(jnp.where(d2 <= dmin, kidx, _K), axis=1, keepdims=True)


def kernel(input_values, ln_gamma, ln_beta, proj_W, code_book, out_W, out_b):
    _cols_pad, overlay_vals, overlay_mask = _CONSTS

    tout, lab = pl.pallas_call(
        _tc_body,
        out_shape=(
            jax.ShapeDtypeStruct((_NPAD, 1), jnp.float32),
            jax.ShapeDtypeStruct((_NPAD, 1), jnp.int32),
        ),
        scratch_shapes=[pltpu.VMEM((_NPAD, _D), jnp.float32)],
    )(input_values, ln_gamma.reshape(1, _D), ln_beta.reshape(1, _D), proj_W,
      code_book, out_W.reshape(1, _D), out_b.reshape(1, 1), overlay_vals,
      overlay_mask)

    return (tout[:_N], lab[:_N, 0])


# speedup vs baseline: 3.1574x; 1.0289x over previous
import functools

import numpy as np
import jax
import jax.numpy as jnp
from jax import lax
from jax.experimental import pallas as pl
from jax.experimental.pallas import tpu as pltpu

_B, _T, _D = 1, 4096, 600
_H, _K = 64, 1024
_MASK_PROB = 0.1
_MASK_TIME = 400
_NUM_MASKS = 5
_SEED = 42
_N = 410
_NPAD = 416


@functools.lru_cache(maxsize=1)
def _consts():
    with jax.default_device(jax.devices("cpu")[0]):
        return _consts_impl()


def _consts_impl():
    k_mask = jax.random.fold_in(jax.random.key(_SEED), 0)
    logits = jax.random.uniform(k_mask, (_B, _T))
    randperm = jnp.argsort(logits, axis=-1).astype(jnp.float32)
    mask = randperm < (_T * _MASK_PROB)
    _rows, cols = jnp.nonzero(mask, size=_B * _N)
    cols = np.asarray(cols).astype(np.int32)

    rng = np.random.RandomState(_SEED)
    k_noise = jax.random.fold_in(jax.random.key(_SEED), 1)
    overlay_vals = np.zeros((_NPAD, _D), np.float32)
    overlay_mask = np.zeros((1, _D), np.float32)
    for i in range(_NUM_MASKS):
        idx = int(rng.randint(0, _D + 1))
        if idx + _MASK_TIME <= _D:
            noise = 0.1 * jax.random.normal(
                jax.random.fold_in(k_noise, i), (_N, _MASK_TIME), dtype=jnp.float32)
            overlay_vals[:_N, idx:idx + _MASK_TIME] = np.asarray(noise)
            overlay_mask[0, idx:idx + _MASK_TIME] = 1.0

    cols_pad = np.concatenate([cols, np.full((_NPAD - _N,), cols[-1], np.int32)])
    return (jnp.asarray(cols_pad), jnp.asarray(overlay_vals),
            jnp.asarray(overlay_mask))


_CONSTS = _consts()


def _tc_body(x_ref, g_ref, b_ref, w_ref, cb_ref, ow_ref, ob_ref, ov_ref,
             om_ref, tout_ref, lab_ref, xg_ref, sem):
    cols = np.asarray(_CONSTS[0])
    descs = []
    for i in range(_NPAD):
        col = int(cols[i])
        descs.append(pltpu.make_async_copy(
            x_ref.at[0, pl.ds(col, 1), :], xg_ref.at[pl.ds(i, 1), :], sem))
    for d in descs:
        d.start()
    for d in descs:
        d.wait()

    x = xg_ref[:]
    mu = jnp.mean(x, axis=1, keepdims=True)
    var = jnp.mean((x - mu) ** 2, axis=1, keepdims=True)
    y = (x - mu) / jnp.sqrt(var + 1e-5) * g_ref[:] + b_ref[:]

    ym = jnp.where(om_ref[:] > 0.5, ov_ref[:], y)
    tout_ref[:] = (jnp.sum(ym * ow_ref[:], axis=1, keepdims=True) + ob_ref[0, 0])

    t = jax.lax.dot_general(y.astype(jnp.bfloat16),
                            w_ref[:].astype(jnp.bfloat16),
                            (((1,), (1,)), ((), ())),
                            preferred_element_type=jnp.float32)
    hi = jax.lax.Precision.HIGHEST
    cb = cb_ref[:]
    tc = jax.lax.dot_general(t, cb, (((1,), (0,)), ((), ())),
                             precision=hi, preferred_element_type=jnp.float32)
    d2 = (jnp.sum(t * t, axis=1, keepdims=True) - 2.0 * tc
          + jnp.sum(cb * cb, axis=0, keepdims=True))
    dmin = jnp.min(d2, axis=1, keepdims=True)
    kidx = jax.lax.broadcasted_iota(jnp.int32, d2.shape, 1)
    lab_ref[:] = jnp.min(jnp.where(d2 <= dmin, kidx, _K), axis=1, keepdims=True)


def kernel(input_values, ln_gamma, ln_beta, proj_W, code_book, out_W, out_b):
    _cols_pad, overlay_vals, overlay_mask = _CONSTS

    tout, lab = pl.pallas_call(
        _tc_body,
        out_shape=(
            jax.ShapeDtypeStruct((_NPAD, 1), jnp.float32),
            jax.ShapeDtypeStruct((_NPAD, 1), jnp.int32),
        ),
        in_specs=[pl.BlockSpec(memory_space=pltpu.MemorySpace.HBM)] + [
            pl.BlockSpec(memory_space=pltpu.MemorySpace.VMEM) for _ in range(8)],
        scratch_shapes=[pltpu.VMEM((_NPAD, _D), jnp.float32),
                        pltpu.SemaphoreType.DMA],
    )(input_values, ln_gamma.reshape(1, _D), ln_beta.reshape(1, _D), proj_W,
      code_book, out_W.reshape(1, _D), out_b.reshape(1, 1), overlay_vals,
      overlay_mask)

    return (tout[:_N], lab[:_N, 0])
